# baseline (speedup 1.0000x reference)
import functools

import jax
import jax.numpy as jnp
from jax import lax
from jax.experimental import pallas as pl
from jax.experimental.pallas import tpu as pltpu
from jax.experimental.pallas import tpu_sc as plsc

B, N, D = 64, 4096, 64
NC, NS, L = 2, 16, 16
BPC = B // NC
BPT = 2
NV = N // L
DG = 8
NH = N // 2
UNITS = BPC // 2
PAD = 8


def _make_resample_kernel():
    mesh = plsc.VectorSubcoreMesh(core_axis_name="c", subcore_axis_name="s")

    @functools.partial(
        pl.kernel,
        mesh=mesh,
        compiler_params=pltpu.CompilerParams(
            use_tc_tiling_on_sc=True, needs_layout_passes=False),
        out_type=jax.ShapeDtypeStruct((B, D, N), jnp.float32),
        scratch_types=[
            pltpu.VMEM((PAD + N,), jnp.float32),
            pltpu.VMEM((N,), jnp.int32),
            pltpu.VMEM((64,), jnp.int32),
            pltpu.VMEM((2, N), jnp.int32),
            pltpu.VMEM((2, DG, N), jnp.float32),
            pltpu.VMEM((2, DG, NH), jnp.float32),
            pltpu.VMEM_SHARED((BPC, N), jnp.int32),
            pltpu.SemaphoreType.DMA,
            pltpu.SemaphoreType.DMA,
            pltpu.SemaphoreType.DMA,
            pltpu.SemaphoreType.DMA,
            pltpu.SemaphoreType.DMA,
        ],
    )
    def resample_kernel(state_hbm, cum_hbm, mask_hbm, out_hbm,
                        cum_v, idxb, mask_v, idx_v, in_v, out_v, idx_sh,
                        sem_in0, sem_in1, sem_idx0, sem_idx1, sem_out):
        cid = lax.axis_index("c")
        sid = lax.axis_index("s")
        iota16 = lax.iota(jnp.int32, L)
        sem_in = (sem_in0, sem_in1)
        sem_idx = (sem_idx0, sem_idx1)

        dgroup = lax.rem(sid, 8)
        half = sid // 8
        d0 = dgroup * DG

        def batch_of_unit(u):
            return cid * BPC + 2 * u + half

        def in_desc(u, p, sem):
            b = batch_of_unit(u)
            return pltpu.make_async_copy(
                state_hbm.at[b, pl.ds(d0, DG)], in_v.at[p], sem)

        def idx_desc(u, p, sem):
            lb = 2 * u + half
            return pltpu.make_async_copy(idx_sh.at[lb], idx_v.at[p], sem)

        def mask_scalar(b):
            grp = b // L
            lane = lax.rem(b, L)
            mv = mask_v[pl.ds(grp * L, L)]
            return jnp.max(jnp.where(iota16 == lane, mv, 0))

        pltpu.sync_copy(mask_hbm, mask_v)
        in_desc(0, 0, sem_in0).start()
        in_desc(1, 1, sem_in1).start()

        zero16 = jnp.zeros((L,), jnp.int32)
        for q in range(BPT):
            lb = sid * BPT + q
            b = cid * BPC + lb

            @pl.when(mask_scalar(b) != 0)
            def _():
                pltpu.sync_copy(cum_hbm.at[pl.ds(b * N, N)],
                                cum_v.at[pl.ds(PAD, N)])
                head = cum_v[pl.ds(0, L)]
                cum_v[pl.ds(0, L)] = jnp.where(
                    iota16 < PAD, jnp.float32(-1.0), head)

                @plsc.parallel_loop(0, NV, unroll=8)
                def _(k):
                    idxb[pl.ds(k * L, L)] = zero16

                def cnt_of(c):
                    x = c * jnp.float32(N) - jnp.float32(0.5)
                    t = x.astype(jnp.int32)
                    return jnp.minimum(jnp.where(x >= 0, t + 1, 0), N)

                @plsc.parallel_loop(0, NV, unroll=4)
                def _(k):
                    hi = cnt_of(cum_v[pl.ds(PAD + k * L, L)])
                    lo = cnt_of(cum_v[pl.ds(PAD - 1 + k * L, L)])
                    vals = iota16 + k * L
                    plsc.store_scatter(idxb, [lo], vals, mask=hi > lo)

                last = cnt_of(cum_v[pl.ds(PAD + N - L, L)])
                p = jnp.max(last)
                p_vec = jnp.full((L,), p, jnp.int32)
                plsc.store_scatter(
                    idxb, [p_vec], jnp.full((L,), N - 1, jnp.int32),
                    mask=(iota16 == 0) & (p_vec < N))

                vm0 = plsc.cummax(idxb[pl.ds(0, L)])
                idxb[pl.ds(0, L)] = vm0

                def pbody(k, prev):
                    vm = plsc.cummax(idxb[pl.ds(k * L, L)])
                    vm = jnp.maximum(
                        vm, jnp.full((L,), prev[L - 1], jnp.int32))
                    idxb[pl.ds(k * L, L)] = vm
                    return vm
                lax.fori_loop(1, NV, pbody, vm0)

                pltpu.sync_copy(idxb, idx_sh.at[lb])

        plsc.subcore_barrier()

        def out_desc(u, src, n0):
            b = batch_of_unit(u)
            return pltpu.make_async_copy(
                src, out_hbm.at[b, pl.ds(d0, DG), pl.ds(n0, NH)], sem_out)

        def drain_out():
            pltpu.make_async_copy(
                in_v.at[0], out_hbm.at[0, pl.ds(0, DG)], sem_out).wait()

        idx_desc(0, 0, sem_idx0).start()
        idx_desc(1, 1, sem_idx1).start()

        def unit(u, p):
            np_ = 1 - p
            in_desc(u, p, sem_in[p]).wait()
            idx_desc(u, p, sem_idx[p]).wait()

            @pl.when(u > 0)
            def _():
                drain_out()

            @pl.when((u >= 1) & (u + 1 < UNITS))
            def _():
                in_desc(u + 1, np_, sem_in[np_]).start()
                idx_desc(u + 1, np_, sem_idx[np_]).start()

            m = mask_scalar(batch_of_unit(u))
            src = in_v.at[p]

            @pl.when(m != 0)
            def _():
                for h in range(2):
                    n0 = h * NH

                    @plsc.parallel_loop(0, NH // L, unroll=4)
                    def _(k):
                        col = idx_v[p, pl.ds(n0 + k * L, L)]
                        for d in range(DG):
                            row = jnp.full((L,), d, jnp.int32)
                            out_v[h, d, pl.ds(k * L, L)] = plsc.load_gather(
                                src, [row, col])
                    out_desc(u, out_v.at[h], n0).start()

            @pl.when(m == 0)
            def _():
                b = batch_of_unit(u)
                pltpu.make_async_copy(
                    src, out_hbm.at[b, pl.ds(d0, DG)], sem_out).start()

        def p2body(g, c):
            unit(2 * g, 0)
            unit(2 * g + 1, 1)
            return c
        lax.fori_loop(0, UNITS // 2, p2body, 0)
        drain_out()

    return resample_kernel


_resample = _make_resample_kernel()


def kernel(state, weight):
    w = weight / jnp.sum(weight, axis=-1, keepdims=True)
    ess = 1.0 / jnp.sum(w * w, axis=-1)
    resample_mask = lax.stop_gradient(ess < 0.75 * N)

    cum = jnp.cumsum(w, axis=-1)
    state_t = jnp.swapaxes(state, 1, 2)
    out_t = _resample(state_t, cum.reshape(B * N),
                      resample_mask.astype(jnp.int32))
    out_state = jnp.swapaxes(out_t, 1, 2)
    out_weight = jnp.where(resample_mask[:, None], jnp.float32(1.0 / N), weight)
    return out_state, out_weight


# speedup vs baseline: 1.0140x; 1.0140x over previous
import functools

import jax
import jax.numpy as jnp
from jax import lax
from jax.experimental import pallas as pl
from jax.experimental.pallas import tpu as pltpu
from jax.experimental.pallas import tpu_sc as plsc

B, N, D = 64, 4096, 64
NC, NS, L = 2, 16, 16
HB = B // 2
BPC = HB // NC
NV = N // L
DG = 8
NH = N // 2
UNITS = BPC // 2
PAD = 8


def _make_resample_kernel(call_idx):
    mesh = plsc.VectorSubcoreMesh(core_axis_name="c", subcore_axis_name="s")
    g0 = call_idx * HB

    @functools.partial(
        pl.kernel,
        mesh=mesh,
        compiler_params=pltpu.CompilerParams(
            use_tc_tiling_on_sc=True, needs_layout_passes=False),
        out_type=(),
        scratch_types=[
            pltpu.VMEM((PAD + N,), jnp.float32),
            pltpu.VMEM((N,), jnp.int32),
            pltpu.VMEM((64,), jnp.int32),
            pltpu.VMEM((2, N), jnp.int32),
            pltpu.VMEM((2, DG, N), jnp.float32),
            pltpu.VMEM((2, DG, NH), jnp.float32),
            pltpu.VMEM_SHARED((BPC, N), jnp.int32),
            pltpu.SemaphoreType.DMA,
            pltpu.SemaphoreType.DMA,
            pltpu.SemaphoreType.DMA,
            pltpu.SemaphoreType.DMA,
            pltpu.SemaphoreType.DMA,
        ],
    )
    def resample_kernel(state_hbm, cum_hbm, mask_hbm, out_hbm,
                        cum_v, idxb, mask_v, idx_v, in_v, out_v, idx_sh,
                        sem_in0, sem_in1, sem_idx0, sem_idx1, sem_out):
        cid = lax.axis_index("c")
        sid = lax.axis_index("s")
        iota16 = lax.iota(jnp.int32, L)
        sem_in = (sem_in0, sem_in1)
        sem_idx = (sem_idx0, sem_idx1)

        dgroup = lax.rem(sid, 8)
        half = sid // 8
        d0 = dgroup * DG

        def batch_of_unit(u):
            return cid * BPC + 2 * u + half

        def in_desc(u, p, sem):
            b = g0 + batch_of_unit(u)
            return pltpu.make_async_copy(
                state_hbm.at[b, pl.ds(d0, DG)], in_v.at[p], sem)

        def idx_desc(u, p, sem):
            lb = 2 * u + half
            return pltpu.make_async_copy(idx_sh.at[lb], idx_v.at[p], sem)

        def mask_scalar(bg):
            grp = bg // L
            lane = lax.rem(bg, L)
            mv = mask_v[pl.ds(grp * L, L)]
            return jnp.max(jnp.where(iota16 == lane, mv, 0))

        pltpu.sync_copy(mask_hbm, mask_v)
        in_desc(0, 0, sem_in0).start()
        in_desc(1, 1, sem_in1).start()

        zero16 = jnp.zeros((L,), jnp.int32)
        lb = sid
        bl = cid * BPC + lb

        @pl.when(mask_scalar(g0 + bl) != 0)
        def _():
            pltpu.sync_copy(cum_hbm.at[pl.ds(bl * N, N)],
                            cum_v.at[pl.ds(PAD, N)])
            head = cum_v[pl.ds(0, L)]
            cum_v[pl.ds(0, L)] = jnp.where(
                iota16 < PAD, jnp.float32(-1.0), head)

            @plsc.parallel_loop(0, NV, unroll=8)
            def _(k):
                idxb[pl.ds(k * L, L)] = zero16

            def cnt_of(c):
                x = c * jnp.float32(N) - jnp.float32(0.5)
                t = x.astype(jnp.int32)
                return jnp.minimum(jnp.where(x >= 0, t + 1, 0), N)

            @plsc.parallel_loop(0, NV, unroll=4)
            def _(k):
                hi = cnt_of(cum_v[pl.ds(PAD + k * L, L)])
                lo = cnt_of(cum_v[pl.ds(PAD - 1 + k * L, L)])
                vals = iota16 + k * L
                plsc.store_scatter(idxb, [lo], vals, mask=hi > lo)

            last = cnt_of(cum_v[pl.ds(PAD + N - L, L)])
            p = jnp.max(last)
            p_vec = jnp.full((L,), p, jnp.int32)
            plsc.store_scatter(
                idxb, [p_vec], jnp.full((L,), N - 1, jnp.int32),
                mask=(iota16 == 0) & (p_vec < N))

            vm0 = plsc.cummax(idxb[pl.ds(0, L)])
            idxb[pl.ds(0, L)] = vm0

            def pbody(k, prev):
                vm = plsc.cummax(idxb[pl.ds(k * L, L)])
                vm = jnp.maximum(
                    vm, jnp.full((L,), prev[L - 1], jnp.int32))
                idxb[pl.ds(k * L, L)] = vm
                return vm
            lax.fori_loop(1, NV, pbody, vm0)

            pltpu.sync_copy(idxb, idx_sh.at[lb])

        plsc.subcore_barrier()

        def out_desc(u, src, n0):
            b = g0 + batch_of_unit(u)
            return pltpu.make_async_copy(
                src, out_hbm.at[b, pl.ds(d0, DG), pl.ds(n0, NH)], sem_out)

        def drain_out():
            pltpu.make_async_copy(
                in_v.at[0], out_hbm.at[0, pl.ds(0, DG)], sem_out).wait()

        idx_desc(0, 0, sem_idx0).start()
        idx_desc(1, 1, sem_idx1).start()

        def unit(u, p):
            np_ = 1 - p
            in_desc(u, p, sem_in[p]).wait()
            idx_desc(u, p, sem_idx[p]).wait()

            @pl.when(u > 0)
            def _():
                drain_out()

            @pl.when((u >= 1) & (u + 1 < UNITS))
            def _():
                in_desc(u + 1, np_, sem_in[np_]).start()
                idx_desc(u + 1, np_, sem_idx[np_]).start()

            m = mask_scalar(g0 + batch_of_unit(u))
            src = in_v.at[p]

            @pl.when(m != 0)
            def _():
                for h in range(2):
                    n0 = h * NH

                    @plsc.parallel_loop(0, NH // L, unroll=4)
                    def _(k):
                        col = idx_v[p, pl.ds(n0 + k * L, L)]
                        for d in range(DG):
                            row = jnp.full((L,), d, jnp.int32)
                            out_v[h, d, pl.ds(k * L, L)] = plsc.load_gather(
                                src, [row, col])
                    out_desc(u, out_v.at[h], n0).start()

            @pl.when(m == 0)
            def _():
                b = g0 + batch_of_unit(u)
                pltpu.make_async_copy(
                    src, out_hbm.at[b, pl.ds(d0, DG)], sem_out).start()

        def p2body(g, c):
            unit(2 * g, 0)
            unit(2 * g + 1, 1)
            return c
        lax.fori_loop(0, UNITS // 2, p2body, 0)
        drain_out()

    return resample_kernel


_resample_lo = _make_resample_kernel(0)
_resample_hi = _make_resample_kernel(1)


def kernel(state, weight):
    w = weight / jnp.sum(weight, axis=-1, keepdims=True)
    ess = 1.0 / jnp.sum(w * w, axis=-1)
    resample_mask = lax.stop_gradient(ess < 0.75 * N)
    mask_i32 = resample_mask.astype(jnp.int32)

    cum_lo = jnp.cumsum(w[:HB], axis=-1).reshape(HB * N)
    cum_hi = jnp.cumsum(w[HB:], axis=-1).reshape(HB * N)

    state_t = jnp.swapaxes(state, 1, 2)
    out_ref = pl.empty_ref_like(pltpu.HBM((B, D, N), jnp.float32))
    _resample_lo(state_t, cum_lo, mask_i32, out_ref)
    _resample_hi(state_t, cum_hi, mask_i32, out_ref)
    out_state = jnp.swapaxes(out_ref[...], 1, 2)
    out_weight = jnp.where(resample_mask[:, None], jnp.float32(1.0 / N), weight)
    return out_state, out_weight
